# baseline (speedup 1.0000x reference)
import functools

import jax
import jax.numpy as jnp
from jax import lax
from jax.experimental import pallas as pl
from jax.experimental.pallas import tpu as pltpu
from jax.experimental.pallas import tpu_sc as plsc

B = 16384
D = 1000
KTC = 600
NC = 2
NS = 16
NW = NC * NS
COLS_PER_W = B // NW
RCHUNK = 120
CHUNKS = [(KTC + c * RCHUNK, min(RCHUNK, D - KTC - c * RCHUNK))
          for c in range((D - KTC + RCHUNK - 1) // RCHUNK)]
GROUPS = COLS_PER_W // 16
UNROLL = 4

_mesh = plsc.VectorSubcoreMesh(core_axis_name="c", subcore_axis_name="s")


@functools.partial(
    pl.kernel,
    out_type=jax.ShapeDtypeStruct((D, B), jnp.float32),
    mesh=_mesh,
    scratch_types=[
        pltpu.VMEM((COLS_PER_W,), jnp.int32),
        pltpu.VMEM((RCHUNK, COLS_PER_W), jnp.float32),
        pltpu.VMEM((RCHUNK, COLS_PER_W), jnp.float32),
        pltpu.SemaphoreType.DMA,
        pltpu.SemaphoreType.DMA,
    ],
    compiler_params=pltpu.CompilerParams(needs_layout_passes=False),
)
def _onehot_sc(x_hbm, out_hbm, idx_v, buf0, buf1, sem0, sem1):
    wid = lax.axis_index("s") * NC + lax.axis_index("c")
    col_base = wid * COLS_PER_W
    pltpu.sync_copy(x_hbm.at[pl.ds(col_base, COLS_PER_W)], idx_v)

    bufs = (buf0, buf1)
    sems = (sem0, sem1)
    z16 = jnp.zeros((16,), jnp.float32)
    ones16 = jnp.ones((16,), jnp.float32)
    iota16 = lax.iota(jnp.int32, 16)

    def _zero(buf):
        def _zero_row(r, carry):
            for j in range(COLS_PER_W // 16):
                buf[r, pl.ds(j * 16, 16)] = z16
            return carry

        lax.fori_loop(0, RCHUNK, _zero_row, 0)

    def _scatter(buf, r0, size, val):
        usize = jnp.full((16,), size, jnp.uint32)

        def _step(i, carry):
            for k in range(UNROLL):
                off = (i * UNROLL + k) * 16
                x16 = idx_v[pl.ds(off, 16)]
                rows = x16 - r0
                mask = plsc.bitcast(rows, jnp.uint32) < usize
                cols = iota16 + off
                plsc.store_scatter(buf, [rows, cols], val, mask=mask)
            return carry

        lax.fori_loop(0, GROUPS // UNROLL, _step, 0)

    def _start(b, r0, size):
        return pltpu.async_copy(
            bufs[b].at[pl.ds(0, size)],
            out_hbm.at[pl.ds(r0, size), pl.ds(col_base, COLS_PER_W)],
            sems[b],
        )

    _zero(bufs[0])
    r0, size = CHUNKS[0]
    _scatter(bufs[0], r0, size, ones16)
    handles = [_start(0, r0, size), None]
    pending = [(r0, size), None]
    _zero(bufs[1])

    for i in range(1, len(CHUNKS)):
        b = i % 2
        if pending[b] is not None:
            handles[b].wait()
            pr0, psize = pending[b]
            _scatter(bufs[b], pr0, psize, z16)
        r0, size = CHUNKS[i]
        _scatter(bufs[b], r0, size, ones16)
        handles[b] = _start(b, r0, size)
        pending[b] = (r0, size)

    handles[0].wait()
    if handles[1] is not None:
        handles[1].wait()


RB = 120
CB = 2048


def _tc_body(x_ref, _, o_ref):
    r = pl.program_id(0)
    rows = lax.broadcasted_iota(jnp.int32, (RB, CB), 0) + r * RB
    o_ref[...] = (rows == x_ref[...][None, :]).astype(jnp.float32)


def _tc_fill(x, buf):
    return pl.pallas_call(
        _tc_body,
        grid=(KTC // RB, B // CB),
        in_specs=[
            pl.BlockSpec((CB,), lambda r, c: (c,)),
            pl.BlockSpec(memory_space=pl.ANY),
        ],
        out_specs=pl.BlockSpec((RB, CB), lambda r, c: (r, c)),
        out_shape=jax.ShapeDtypeStruct((D, B), jnp.float32),
        input_output_aliases={1: 0},
    )(x, buf)


def kernel(x):
    xi = x.astype(jnp.int32)
    return _tc_fill(xi, _onehot_sc(xi)).T
